# Initial kernel scaffold; baseline (speedup 1.0000x reference)
import jax
import jax.numpy as jnp
from jax.experimental import pallas as pl


def kernel(targets, contexts, negatives, in_emb, out_emb):
    raise NotImplementedError("write your pallas kernel here")


# speedup vs baseline: 10.5210x; 10.5210x over previous
import functools

import jax
import jax.numpy as jnp
from jax import lax
from jax.experimental import pallas as pl
from jax.experimental.pallas import tpu as pltpu
from jax.experimental.pallas import tpu_sc as plsc

B = 16384
K = 20
D = 128
DC = D // 16

NC, NS, L = 2, 16, 16
NW = NC * NS
BPW = B // NW
CB = 16
NCHUNK = BPW // CB
NEG_PER_CHUNK = CB * K
IDX_COLS = 64
IDX_ROWS_PER_CHUNK = NEG_PER_CHUNK // IDX_COLS
IDX_ROWS_PER_W = BPW * K // IDX_COLS


def _log_sigmoid(x):
    u = jnp.exp(-jnp.abs(x))
    z = u / (u + 2.0)
    z2 = z * z
    p = 1.0 + z2 * (
        (1.0 / 3.0) + z2 * ((1.0 / 5.0) + z2 * ((1.0 / 7.0) + z2 * (1.0 / 9.0)))
    )
    return jnp.minimum(x, 0.0) - 2.0 * z * p


def _reduce16(vecs, lane):
    for w in (8, 4, 2, 1):
        perm = lane ^ w
        low = (lane & w) == 0
        nxt = []
        for i in range(0, len(vecs), 2):
            a, b = vecs[i], vecs[i + 1]
            a_s = a.at[perm].get(mode="promise_in_bounds")
            b_s = b.at[perm].get(mode="promise_in_bounds")
            nxt.append(jnp.where(low, a + a_s, b + b_s))
        vecs = nxt
    return vecs[0]


def _sgns_kernel(targets_h, contexts_h, negatives_h, in_emb_h, out_emb_h,
                 out_h, idx_t, idx_c, idx_n,
                 vt_buf0, vc_buf0, vn_buf0, sem0,
                 vt_buf1, vc_buf1, vn_buf1, sem1,
                 pos_stage, neg_stage, total):
    wid = lax.axis_index("s") * NC + lax.axis_index("c")
    wbase = wid * BPW

    pltpu.sync_copy(targets_h.at[pl.ds(wbase, BPW)], idx_t)
    pltpu.sync_copy(contexts_h.at[pl.ds(wbase, BPW)], idx_c)
    pltpu.sync_copy(negatives_h.at[pl.ds(wid * IDX_ROWS_PER_W, IDX_ROWS_PER_W)],
                    idx_n)

    total[...] = jnp.zeros((L,), jnp.float32)
    lane = lax.iota(jnp.int32, L)
    slots = ((vt_buf0, vc_buf0, vn_buf0, sem0),
             (vt_buf1, vc_buf1, vn_buf1, sem1))

    def issue(c, slot):
        vt_b, vc_b, vn_b, sem = slot
        pltpu.async_copy(in_emb_h.at[idx_t[pl.ds(c * CB, CB)]], vt_b, sem)
        pltpu.async_copy(out_emb_h.at[idx_c[pl.ds(c * CB, CB)]], vc_b, sem)
        for j in range(IDX_ROWS_PER_CHUNK):
            pltpu.async_copy(
                out_emb_h.at[idx_n.at[c * IDX_ROWS_PER_CHUNK + j]],
                vn_b.at[pl.ds(j * IDX_COLS, IDX_COLS)], sem)

    def drain(slot):
        vt_b, vc_b, vn_b, sem = slot
        pltpu.make_async_copy(in_emb_h.at[pl.ds(0, CB)], vt_b, sem).wait()
        pltpu.make_async_copy(out_emb_h.at[pl.ds(0, CB)], vc_b, sem).wait()
        for j in range(IDX_ROWS_PER_CHUNK):
            pltpu.make_async_copy(
                out_emb_h.at[pl.ds(0, IDX_COLS)],
                vn_b.at[pl.ds(j * IDX_COLS, IDX_COLS)], sem).wait()

    def compute(slot):
        vt_b, vc_b, vn_b, _ = slot

        def b_body(b, _unused):
            accs = None
            for d in range(DC):
                vtd = vt_b[b, pl.ds(d * L, L)]
                rows = [vc_b[b, pl.ds(d * L, L)]]
                rows += [vn_b[b * K + k, pl.ds(d * L, L)] for k in range(K)]
                if accs is None:
                    accs = [vtd * r for r in rows]
                else:
                    accs = [a + vtd * r for a, r in zip(accs, rows)]
            pos_stage[pl.ds(b * L, L)] = accs[0]
            for k in range(K):
                neg_stage[pl.ds((b * K + k) * L, L)] = accs[k + 1]
            return 0

        lax.fori_loop(0, CB, b_body, 0)

        tot = total[...]
        rows = [pos_stage[pl.ds(p * L, L)] for p in range(L)]
        tot = tot + _log_sigmoid(_reduce16(rows, lane))
        for g in range(K):
            rows = [neg_stage[pl.ds((g * L + p) * L, L)] for p in range(L)]
            tot = tot + _log_sigmoid(-_reduce16(rows, lane))
        total[...] = tot

    issue(0, slots[0])

    def pair_body(i, _unused):
        c0 = i * 2
        issue(c0 + 1, slots[1])
        drain(slots[0])
        compute(slots[0])

        @pl.when(i < NCHUNK // 2 - 1)
        def _():
            issue(c0 + 2, slots[0])

        drain(slots[1])
        compute(slots[1])
        return 0

    lax.fori_loop(0, NCHUNK // 2, pair_body, 0)
    pltpu.sync_copy(total, out_h.at[wid])


@jax.jit
def _sgns(targets, contexts, negatives, in_emb, out_emb):
    mesh = plsc.VectorSubcoreMesh(core_axis_name="c", subcore_axis_name="s")
    partials = pl.kernel(
        _sgns_kernel,
        mesh=mesh,
        out_type=jax.ShapeDtypeStruct((NW, L), jnp.float32),
        scratch_types=[
            pltpu.VMEM((BPW,), jnp.int32),
            pltpu.VMEM((BPW,), jnp.int32),
            pltpu.VMEM((IDX_ROWS_PER_W, IDX_COLS), jnp.int32),
            pltpu.VMEM((CB, D), jnp.float32),
            pltpu.VMEM((CB, D), jnp.float32),
            pltpu.VMEM((NEG_PER_CHUNK, D), jnp.float32),
            pltpu.SemaphoreType.DMA,
            pltpu.VMEM((CB, D), jnp.float32),
            pltpu.VMEM((CB, D), jnp.float32),
            pltpu.VMEM((NEG_PER_CHUNK, D), jnp.float32),
            pltpu.SemaphoreType.DMA,
            pltpu.VMEM((CB * L,), jnp.float32),
            pltpu.VMEM((NEG_PER_CHUNK * L,), jnp.float32),
            pltpu.VMEM((L,), jnp.float32),
        ],
    )(targets, contexts, negatives.reshape(B * K // IDX_COLS, IDX_COLS),
      in_emb, out_emb)
    return -jnp.sum(partials) / B


def kernel(targets, contexts, negatives, in_emb, out_emb):
    return _sgns(targets, contexts, negatives, in_emb, out_emb)
